# baseline (speedup 1.0000x reference)
import numpy as np

import jax
import jax.numpy as jnp
from jax.experimental import pallas as pl

B, T, DIN, DH, DM, K = 16, 4096, 32, 64, 128, 32
SEG = T // K
NB = 16
GRID = B // NB


_POOL = np.zeros((T, K), np.float32)
_POOL[np.arange(T), np.arange(T) // SEG] = 1.0 / SEG


def _mlp_pool_kernel(x_ref, w1_ref, b1_ref, w2_ref, b2_ref, g_ref, p_ref, o_ref):
    x = x_ref[...].reshape(NB * DIN, T)
    w1 = w1_ref[...]
    zc = jnp.zeros((DIN, DH), jnp.float32)
    wbd = jnp.concatenate(
        [
            jnp.concatenate([w1 if r == c else zc for c in range(NB)], axis=1)
            for r in range(NB)
        ],
        axis=0,
    )
    b1t = jnp.concatenate([b1_ref[...]] * NB, axis=0)
    h = jax.lax.dot_general(
        wbd, x, (((0,), (0,)), ((), ())), preferred_element_type=jnp.float32
    ) + b1t
    h = 0.5 * h * (1.0 + jax.lax.erf(h * jnp.float32(0.7071067811865476)))
    s = jnp.dot(h, p_ref[...], preferred_element_type=jnp.float32)
    scale = jnp.tanh(g_ref[0, 0])
    for bi in range(NB):
        y = jax.lax.dot_general(
            s[bi * DH : (bi + 1) * DH],
            w2_ref[...],
            (((0,), (0,)), ((), ())),
            preferred_element_type=jnp.float32,
        )
        o_ref[bi] = (y + b2_ref[...]) * scale


def kernel(imu_seq, W1, b1, W2, b2, gate):
    xt = jnp.swapaxes(imu_seq, 1, 2)
    b1r = b1.reshape(DH, 1)
    b2r = b2.reshape(1, DM)
    gr = gate.reshape(1, 1)
    out = pl.pallas_call(
        _mlp_pool_kernel,
        grid=(GRID,),
        in_specs=[
            pl.BlockSpec((NB, DIN, T), lambda g: (g, 0, 0)),
            pl.BlockSpec((DIN, DH), lambda g: (0, 0)),
            pl.BlockSpec((DH, 1), lambda g: (0, 0)),
            pl.BlockSpec((DH, DM), lambda g: (0, 0)),
            pl.BlockSpec((1, DM), lambda g: (0, 0)),
            pl.BlockSpec((1, 1), lambda g: (0, 0)),
            pl.BlockSpec((T, K), lambda g: (0, 0)),
        ],
        out_specs=pl.BlockSpec((NB, K, DM), lambda g: (g, 0, 0)),
        out_shape=jax.ShapeDtypeStruct((B, K, DM), jnp.float32),
    )(xt, W1, b1r, W2, b2r, gr, jnp.asarray(_POOL))
    return out


# speedup vs baseline: 1.1721x; 1.1721x over previous
import numpy as np

import jax
import jax.numpy as jnp
from jax.experimental import pallas as pl

B, T, DIN, DH, DM, K = 16, 4096, 32, 64, 128, 32
SEG = T // K
NB = 8
GRID = B // NB
TC = 2
TCHUNK = T // TC
KC = K // TC


_POOL = np.zeros((TC, TCHUNK, KC), np.float32)
for _tc in range(TC):
    _t = np.arange(TCHUNK)
    _POOL[_tc, _t, (_t + _tc * TCHUNK) // SEG - _tc * KC] = 1.0 / SEG


def _mlp_pool_kernel(x_ref, w1_ref, b1_ref, w2_ref, b2_ref, g_ref, p_ref, o_ref):
    x = x_ref[...].reshape(NB * DIN, TCHUNK)
    w1 = w1_ref[...]
    zc = jnp.zeros((DIN, DH), jnp.float32)
    wbd = jnp.concatenate(
        [
            jnp.concatenate([w1 if r == c else zc for c in range(NB)], axis=1)
            for r in range(NB)
        ],
        axis=0,
    )
    b1t = jnp.concatenate([b1_ref[...]] * NB, axis=0)
    h = jax.lax.dot_general(
        wbd, x, (((0,), (0,)), ((), ())), preferred_element_type=jnp.float32
    ) + b1t
    h = 0.5 * h * (1.0 + jax.lax.erf(h * jnp.float32(0.7071067811865476)))
    s = jnp.dot(h, p_ref[0], preferred_element_type=jnp.float32)
    scale = jnp.tanh(g_ref[0, 0])
    for bi in range(NB):
        y = jax.lax.dot_general(
            s[bi * DH : (bi + 1) * DH],
            w2_ref[...],
            (((0,), (0,)), ((), ())),
            preferred_element_type=jnp.float32,
        )
        o_ref[bi] = (y + b2_ref[...]) * scale


def kernel(imu_seq, W1, b1, W2, b2, gate):
    xt = jnp.swapaxes(imu_seq, 1, 2)
    b1r = b1.reshape(DH, 1)
    b2r = b2.reshape(1, DM)
    gr = gate.reshape(1, 1)
    out = pl.pallas_call(
        _mlp_pool_kernel,
        grid=(GRID, TC),
        in_specs=[
            pl.BlockSpec((NB, DIN, TCHUNK), lambda g, tc: (g, 0, tc)),
            pl.BlockSpec((DIN, DH), lambda g, tc: (0, 0)),
            pl.BlockSpec((DH, 1), lambda g, tc: (0, 0)),
            pl.BlockSpec((DH, DM), lambda g, tc: (0, 0)),
            pl.BlockSpec((1, DM), lambda g, tc: (0, 0)),
            pl.BlockSpec((1, 1), lambda g, tc: (0, 0)),
            pl.BlockSpec((1, TCHUNK, KC), lambda g, tc: (tc, 0, 0)),
        ],
        out_specs=pl.BlockSpec((NB, KC, DM), lambda g, tc: (g, tc, 0)),
        out_shape=jax.ShapeDtypeStruct((B, K, DM), jnp.float32),
    )(xt, W1, b1r, W2, b2r, gr, jnp.asarray(_POOL))
    return out


# speedup vs baseline: 1.2390x; 1.0571x over previous
import numpy as np

import jax
import jax.numpy as jnp
from jax.experimental import pallas as pl

B, T, DIN, DH, DM, K = 16, 4096, 32, 64, 128, 32
SEG = T // K
NB = 8
GRID = B // NB


_POOL = np.zeros((T, K), np.float32)
_POOL[np.arange(T), np.arange(T) // SEG] = 1.0 / SEG


def _mlp_pool_kernel(x_ref, w1_ref, b1_ref, w2_ref, b2_ref, g_ref, p_ref, o_ref):
    x = x_ref[...].reshape(NB * DIN, T)
    w1 = w1_ref[...]
    zc = jnp.zeros((DIN, DH), jnp.float32)
    wbd = jnp.concatenate(
        [
            jnp.concatenate([w1 if r == c else zc for c in range(NB)], axis=1)
            for r in range(NB)
        ],
        axis=0,
    )
    b1t = jnp.concatenate([b1_ref[...]] * NB, axis=0)
    h = jax.lax.dot_general(
        wbd, x, (((0,), (0,)), ((), ())), preferred_element_type=jnp.float32
    ) + b1t
    h = 0.5 * h * (1.0 + jax.lax.erf(h * jnp.float32(0.7071067811865476)))
    s = jnp.dot(h, p_ref[...], preferred_element_type=jnp.float32)
    scale = jnp.tanh(g_ref[0, 0])
    for bi in range(NB):
        y = jax.lax.dot_general(
            s[bi * DH : (bi + 1) * DH],
            w2_ref[...],
            (((0,), (0,)), ((), ())),
            preferred_element_type=jnp.float32,
        )
        o_ref[bi] = (y + b2_ref[...]) * scale


def kernel(imu_seq, W1, b1, W2, b2, gate):
    xt = jnp.swapaxes(imu_seq, 1, 2)
    b1r = b1.reshape(DH, 1)
    b2r = b2.reshape(1, DM)
    gr = gate.reshape(1, 1)
    out = pl.pallas_call(
        _mlp_pool_kernel,
        grid=(GRID,),
        in_specs=[
            pl.BlockSpec((NB, DIN, T), lambda g: (g, 0, 0)),
            pl.BlockSpec((DIN, DH), lambda g: (0, 0)),
            pl.BlockSpec((DH, 1), lambda g: (0, 0)),
            pl.BlockSpec((DH, DM), lambda g: (0, 0)),
            pl.BlockSpec((1, DM), lambda g: (0, 0)),
            pl.BlockSpec((1, 1), lambda g: (0, 0)),
            pl.BlockSpec((T, K), lambda g: (0, 0)),
        ],
        out_specs=pl.BlockSpec((NB, K, DM), lambda g: (g, 0, 0)),
        out_shape=jax.ShapeDtypeStruct((B, K, DM), jnp.float32),
    )(xt, W1, b1r, W2, b2r, gr, jnp.asarray(_POOL))
    return out
